# baseline (speedup 1.0000x reference)
import functools

import jax
import jax.numpy as jnp
from jax.experimental import pallas as pl
from jax.experimental.pallas import tpu as pltpu

V, D, M, K = 50000, 512, 4, 512
CHUNK = D // M


def _dpq_body(x_ref, cb_ref, o_ref):
    x = x_ref[:]
    for m in range(M):
        xm = x[:, m, :]
        mx = jnp.max(xm, axis=-1, keepdims=True)
        e = jnp.exp(xm - mx)
        s = jnp.sum(e, axis=-1, keepdims=True)
        cbm = cb_ref[m]
        acc = jax.lax.dot_general(
            e.astype(jnp.bfloat16), cbm.astype(jnp.bfloat16),
            (((1,), (0,)), ((), ())),
            preferred_element_type=jnp.float32,
        )
        o_ref[:, m * CHUNK:(m + 1) * CHUNK] = acc * (1.0 / s)


@functools.partial(jax.jit, static_argnames=("tile_v",))
def _dpq(assign_logits, codebooks, tile_v=400):
    grid = (V // tile_v,)
    return pl.pallas_call(
        _dpq_body,
        grid=grid,
        in_specs=[
            pl.BlockSpec((tile_v, M, K), lambda i: (i, 0, 0)),
            pl.BlockSpec((M, K, CHUNK), lambda i: (0, 0, 0)),
        ],
        out_specs=pl.BlockSpec((tile_v, D), lambda i: (i, 0)),
        out_shape=jax.ShapeDtypeStruct((V, D), jnp.float32),
        compiler_params=pltpu.CompilerParams(
            dimension_semantics=("parallel",),
        ),
    )(assign_logits, codebooks)


def kernel(assign_logits, codebooks):
    return _dpq(assign_logits, codebooks)


# speedup vs baseline: 5.0803x; 5.0803x over previous
import functools

import jax
import jax.numpy as jnp
from jax.experimental import pallas as pl
from jax.experimental.pallas import tpu as pltpu

V, D, M, K = 50000, 512, 4, 512
CHUNK = D // M


def _make_body(tile_v, n_i):
    n_t = n_i * M

    def body(x_hbm, cb_ref, o_ref, xbuf, sem):
        i = pl.program_id(0)
        j = pl.program_id(1)
        t = i * M + j

        def start_copy(step, slot):
            ii = step // M
            jj = step % M
            pltpu.make_async_copy(
                x_hbm.at[pl.ds(ii * tile_v, tile_v), jj],
                xbuf.at[slot],
                sem.at[slot],
            ).start()

        @pl.when(t == 0)
        def _():
            start_copy(0, 0)

        @pl.when(t + 1 < n_t)
        def _():
            start_copy(t + 1, (t + 1) % 2)

        slot = t % 2
        pltpu.make_async_copy(
            x_hbm.at[pl.ds(i * tile_v, tile_v), j],
            xbuf.at[slot],
            sem.at[slot],
        ).wait()

        xm = xbuf[slot]
        mx = jnp.max(xm, axis=-1, keepdims=True)
        e = jnp.exp(xm - mx)
        s = jnp.sum(e, axis=-1, keepdims=True)
        cbm = cb_ref[j]
        acc = jax.lax.dot_general(
            e.astype(jnp.bfloat16), cbm.astype(jnp.bfloat16),
            (((1,), (0,)), ((), ())),
            preferred_element_type=jnp.float32,
        )
        o_ref[:] = acc * (1.0 / s)

    return body


@functools.partial(jax.jit, static_argnames=("tile_v",))
def _dpq(assign_logits, codebooks, tile_v=2000):
    n_i = V // tile_v
    grid = (n_i, M)
    return pl.pallas_call(
        _make_body(tile_v, n_i),
        grid=grid,
        in_specs=[
            pl.BlockSpec(memory_space=pl.ANY),
            pl.BlockSpec((M, K, CHUNK), lambda i, j: (0, 0, 0)),
        ],
        out_specs=pl.BlockSpec((tile_v, CHUNK), lambda i, j: (i, j)),
        out_shape=jax.ShapeDtypeStruct((V, D), jnp.float32),
        scratch_shapes=[
            pltpu.VMEM((2, tile_v, K), jnp.float32),
            pltpu.SemaphoreType.DMA((2,)),
        ],
        compiler_params=pltpu.CompilerParams(
            dimension_semantics=("arbitrary", "arbitrary"),
        ),
    )(assign_logits, codebooks)


def kernel(assign_logits, codebooks):
    return _dpq(assign_logits, codebooks)
